# baseline (speedup 1.0000x reference)
import dataclasses
import functools

import jax
import jax.numpy as jnp
from jax import lax
from jax.experimental import pallas as pl
from jax.experimental.pallas import tpu as pltpu
from jax.experimental.pallas import tpu_sc as plsc

_K = 32
_QB = 256
_WN = 2560
_WPAD = 12800
_M = 8192
_NW = 32
_RPW = _M // _NW
_NCH = _WN // 16
_NSUP = _NCH // 16


def _window(width, s0, q_parts, r_ref):
    qb, qx, qy, qz = q_parts
    rb = r_ref[0:1, pl.ds(s0, width)]
    rx = r_ref[1:2, pl.ds(s0, width)]
    ry = r_ref[2:3, pl.ds(s0, width)]
    rz = r_ref[3:4, pl.ds(s0, width)]
    q2 = qx * qx + qy * qy + qz * qz
    r2 = rx * rx + ry * ry + rz * rz
    bf = jnp.bfloat16
    qmat = jnp.concatenate([qx, qy, qz], axis=1).astype(bf)
    rmat = jnp.concatenate([rx, ry, rz], axis=0).astype(bf)
    qr = jnp.dot(qmat, rmat, preferred_element_type=jnp.float32)
    dist = (q2 + r2) - 2.0 * qr
    return jnp.where(qb != rb, jnp.float32(1e30), dist)


def _tc_dist_block(q_ref, r_ref, d_ref, s_ref):
    q = q_ref[...]
    qb = q[:, 0:1]
    q_parts = (qb, q[:, 1:2], q[:, 2:3], q[:, 3:4])

    b_lo = jnp.min(qb)
    b_hi = jnp.max(qb)
    rb_full = r_ref[0:1, :]
    r_lo = jnp.sum((rb_full < b_lo).astype(jnp.int32))
    r_hi = jnp.sum((rb_full <= b_hi).astype(jnp.int32))
    s0 = (r_lo // 128) * 128
    fits = (r_hi - s0) <= _WN

    @pl.when(fits)
    def _narrow():
        d_ref[...] = _window(_WN, s0, q_parts, r_ref)
        s_ref[...] = jnp.full((_QB, 1), 0, jnp.int32) + s0

    @pl.when(jnp.logical_not(fits))
    def _wide():
        r_mid = jnp.sum((rb_full < b_hi).astype(jnp.int32))
        s1 = (r_mid // 128) * 128
        w_lo = _window(_WN, s0, q_parts, r_ref)
        w_hi = _window(_WN, s1, q_parts, r_ref)
        hi_row = qb == b_hi
        d_ref[...] = jnp.where(hi_row, w_hi, w_lo)
        s_ref[...] = jnp.where(hi_row, s1, s0) + jnp.full(
            (_QB, 1), 0, jnp.int32)


def _sc_select(d_hbm, o_hbm, rowbuf, gmin, outbuf, sem):
    wid = lax.axis_index("s") * 2 + lax.axis_index("c")
    base = wid * _RPW
    lanes = jax.lax.broadcasted_iota(jnp.int32, (16,), 0)
    inf16 = jnp.full((16,), jnp.inf, jnp.float32)

    for g in range(_NCH // 16, 16):
        gmin[pl.ds(16 * g, 16)] = inf16

    @pl.loop(0, _RPW)
    def _row(r):
        row = base + r
        pltpu.sync_copy(d_hbm.at[row], rowbuf)

        for g in range(_NSUP):
            acc = inf16
            for j in range(16):
                v = plsc.load_gather(rowbuf, [256 * g + 16 * lanes + j])
                acc = jnp.minimum(acc, v)
            gmin[pl.ds(16 * g, 16)] = acc

        smin = inf16
        for j in range(16):
            v = plsc.load_gather(gmin, [16 * lanes + j])
            smin = jnp.minimum(smin, v)

        def round_(k, carry):
            smin, o0, o1 = carry
            m = jnp.min(smin)
            l_star = jnp.max(plsc.all_reduce_ffs(smin == m))
            gvec = gmin[pl.ds(16 * l_star, 16)]
            c_off = jnp.max(plsc.all_reduce_ffs(gvec == m))
            c_star = 16 * l_star + c_off
            cv = rowbuf[pl.ds(16 * c_star, 16)]
            j_star = jnp.max(plsc.all_reduce_ffs(cv == m))
            idx = jnp.full((16,), 16 * c_star + j_star, jnp.int32)
            first = k < 16
            hit = lanes == jnp.where(first, k, k - 16)
            o0 = jnp.where(hit & first, idx, o0)
            o1 = jnp.where(hit & jnp.logical_not(first), idx, o1)
            cv2 = jnp.where(lanes == j_star, jnp.float32(jnp.inf), cv)
            rowbuf[pl.ds(16 * c_star, 16)] = cv2
            ng = jnp.min(cv2)
            gv2 = jnp.where(lanes == c_off, ng, gvec)
            gmin[pl.ds(16 * l_star, 16)] = gv2
            ns = jnp.min(gv2)
            return (jnp.where(lanes == l_star, ns, smin), o0, o1)

        zero16 = jnp.zeros((16,), jnp.int32)
        _, o0, o1 = lax.fori_loop(0, _K, round_, (smin, zero16, zero16))
        outbuf[pl.ds(0, 16)] = o0
        outbuf[pl.ds(16, 16)] = o1
        pltpu.sync_copy(outbuf, o_hbm.at[row])


def _sc_select_call(dist):
    mesh = plsc.VectorSubcoreMesh(core_axis_name="c", subcore_axis_name="s")
    cp = pltpu.CompilerParams()
    if "needs_layout_passes" in pltpu.CompilerParams.__dataclass_fields__:
        cp = dataclasses.replace(cp, needs_layout_passes=False)
    kern = functools.partial(
        pl.kernel,
        out_type=jax.ShapeDtypeStruct((_M, _K), jnp.int32),
        mesh=mesh,
        compiler_params=cp,
        scratch_types=[
            pltpu.VMEM((_WN,), jnp.float32),
            pltpu.VMEM((256,), jnp.float32),
            pltpu.VMEM((_K,), jnp.int32),
            pltpu.SemaphoreType.DMA,
        ],
    )(_sc_select)
    return kern(dist)


def kernel(ref_bxyz, query_bxyz):
    m = query_bxyz.shape[0]
    n = ref_bxyz.shape[0]
    rt = jnp.transpose(ref_bxyz)
    rt = jnp.concatenate(
        [rt, jnp.full((4, _WPAD - n), 1e9, jnp.float32)], axis=1)
    rt = jnp.concatenate(
        [rt, jnp.zeros((4, _WPAD), jnp.float32)], axis=0)

    dist, s0 = pl.pallas_call(
        _tc_dist_block,
        grid=(m // _QB,),
        in_specs=[
            pl.BlockSpec((_QB, 4), lambda i: (i, 0)),
            pl.BlockSpec((8, _WPAD), lambda i: (0, 0)),
        ],
        out_specs=[
            pl.BlockSpec((_QB, _WN), lambda i: (i, 0)),
            pl.BlockSpec((_QB, 1), lambda i: (i, 0)),
        ],
        out_shape=[
            jax.ShapeDtypeStruct((m, _WN), jnp.float32),
            jax.ShapeDtypeStruct((m, 1), jnp.int32),
        ],
    )(query_bxyz, rt)

    loc = _sc_select_call(dist)
    e_ref = (loc + s0).reshape(-1)
    e_query = jnp.broadcast_to(
        jnp.arange(m, dtype=jnp.int32)[:, None], (m, _K)).reshape(-1)
    return (e_ref, e_query)


# speedup vs baseline: 1.9420x; 1.9420x over previous
import dataclasses
import functools

import jax
import jax.numpy as jnp
from jax import lax
from jax.experimental import pallas as pl
from jax.experimental.pallas import tpu as pltpu
from jax.experimental.pallas import tpu_sc as plsc

_K = 32
_QB = 256
_WN = 2560
_WS = 4608
_WPAD = 12800
_M = 8192
_SPLIT = 4096
_NW = 32
_NCH = _WN // 16
_NSUP = _NCH // 16


def _window(width, s0, q_parts, r_ref):
    qb, qx, qy, qz = q_parts
    rb = r_ref[0:1, pl.ds(s0, width)]
    rx = r_ref[1:2, pl.ds(s0, width)]
    ry = r_ref[2:3, pl.ds(s0, width)]
    rz = r_ref[3:4, pl.ds(s0, width)]
    q2 = qx * qx + qy * qy + qz * qz
    r2 = rx * rx + ry * ry + rz * rz
    bf = jnp.bfloat16
    qmat = jnp.concatenate([qx, qy, qz], axis=1).astype(bf)
    rmat = jnp.concatenate([rx, ry, rz], axis=0).astype(bf)
    qr = jnp.dot(qmat, rmat, preferred_element_type=jnp.float32)
    dist = (q2 + r2) - 2.0 * qr
    return jnp.where(qb != rb, jnp.float32(1e30), dist)


def _bounds(qb, r_ref):
    b_lo = jnp.min(qb)
    b_hi = jnp.max(qb)
    rb_full = r_ref[0:1, :]
    r_lo = jnp.sum((rb_full < b_lo).astype(jnp.int32))
    r_hi = jnp.sum((rb_full <= b_hi).astype(jnp.int32))
    s0 = (r_lo // 128) * 128
    return b_hi, rb_full, r_hi, s0


def _extract(width, s0, q_parts, r_ref, o_ref, d_ref):
    dist = _window(width, s0, q_parts, r_ref)
    d_ref[:, 0:width] = dist

    iota = jax.lax.broadcasted_iota(jnp.int32, (1, width), 1)
    lane = jax.lax.broadcasted_iota(jnp.int32, (1, _K), 1)

    def body(k, best):
        dmat = d_ref[:, 0:width]
        m = jnp.min(dmat, axis=1, keepdims=True)
        isel = jnp.min(
            jnp.where(dmat == m, iota, jnp.int32(2**31 - 1)),
            axis=1, keepdims=True)
        d_ref[:, 0:width] = jnp.where(
            iota == isel, jnp.float32(jnp.inf), dmat)
        return jnp.where(lane == k, isel, best)

    best = jax.lax.fori_loop(
        0, _K, body, jnp.zeros((_QB, _K), jnp.int32))
    o_ref[...] = best + s0


def _tc_extract_block(q_ref, r_ref, o_ref, d_ref):
    q = q_ref[...]
    qb = q[:, 0:1]
    q_parts = (qb, q[:, 1:2], q[:, 2:3], q[:, 3:4])
    _, _, r_hi, s0 = _bounds(qb, r_ref)
    fits = (r_hi - s0) <= _WN

    @pl.when(fits)
    def _narrow():
        _extract(_WN, s0, q_parts, r_ref, o_ref, d_ref)

    @pl.when(jnp.logical_not(fits))
    def _wide():
        _extract(_WS, s0, q_parts, r_ref, o_ref, d_ref)


def _tc_dist_block(q_ref, r_ref, d_ref, s_ref):
    q = q_ref[...]
    qb = q[:, 0:1]
    q_parts = (qb, q[:, 1:2], q[:, 2:3], q[:, 3:4])
    b_hi, rb_full, r_hi, s0 = _bounds(qb, r_ref)
    fits = (r_hi - s0) <= _WN

    @pl.when(fits)
    def _narrow():
        d_ref[...] = _window(_WN, s0, q_parts, r_ref)
        s_ref[...] = jnp.full((_QB, 1), 0, jnp.int32) + s0

    @pl.when(jnp.logical_not(fits))
    def _wide():
        r_mid = jnp.sum((rb_full < b_hi).astype(jnp.int32))
        s1 = (r_mid // 128) * 128
        w_lo = _window(_WN, s0, q_parts, r_ref)
        w_hi = _window(_WN, s1, q_parts, r_ref)
        hi_row = qb == b_hi
        d_ref[...] = jnp.where(hi_row, w_hi, w_lo)
        s_ref[...] = jnp.where(hi_row, s1, s0) + jnp.full(
            (_QB, 1), 0, jnp.int32)


def _sc_select(rpw, d_hbm, o_hbm, rowbuf, gmin, outbuf, sem):
    wid = lax.axis_index("s") * 2 + lax.axis_index("c")
    base = wid * rpw
    lanes = jax.lax.broadcasted_iota(jnp.int32, (16,), 0)
    inf16 = jnp.full((16,), jnp.inf, jnp.float32)

    for g in range(_NCH // 16, 16):
        gmin[pl.ds(16 * g, 16)] = inf16

    @pl.loop(0, rpw)
    def _row(r):
        row = base + r
        pltpu.sync_copy(d_hbm.at[row], rowbuf)

        for g in range(_NSUP):
            acc = inf16
            for j in range(16):
                v = plsc.load_gather(rowbuf, [256 * g + 16 * lanes + j])
                acc = jnp.minimum(acc, v)
            gmin[pl.ds(16 * g, 16)] = acc

        smin = inf16
        for j in range(16):
            v = plsc.load_gather(gmin, [16 * lanes + j])
            smin = jnp.minimum(smin, v)

        def round_(k, carry):
            smin, o0, o1 = carry
            m = jnp.min(smin)
            l_star = jnp.max(plsc.all_reduce_ffs(smin == m))
            gvec = gmin[pl.ds(16 * l_star, 16)]
            c_off = jnp.max(plsc.all_reduce_ffs(gvec == m))
            c_star = 16 * l_star + c_off
            cv = rowbuf[pl.ds(16 * c_star, 16)]
            j_star = jnp.max(plsc.all_reduce_ffs(cv == m))
            idx = jnp.full((16,), 16 * c_star + j_star, jnp.int32)
            first = k < 16
            hit = lanes == jnp.where(first, k, k - 16)
            o0 = jnp.where(hit & first, idx, o0)
            o1 = jnp.where(hit & jnp.logical_not(first), idx, o1)
            cv2 = jnp.where(lanes == j_star, jnp.float32(jnp.inf), cv)
            rowbuf[pl.ds(16 * c_star, 16)] = cv2
            ng = jnp.min(cv2)
            gv2 = jnp.where(lanes == c_off, ng, gvec)
            gmin[pl.ds(16 * l_star, 16)] = gv2
            ns = jnp.min(gv2)
            return (jnp.where(lanes == l_star, ns, smin), o0, o1)

        zero16 = jnp.zeros((16,), jnp.int32)
        _, o0, o1 = lax.fori_loop(0, _K, round_, (smin, zero16, zero16))
        outbuf[pl.ds(0, 16)] = o0
        outbuf[pl.ds(16, 16)] = o1
        pltpu.sync_copy(outbuf, o_hbm.at[row])


def _sc_select_call(dist):
    rows = dist.shape[0]
    mesh = plsc.VectorSubcoreMesh(core_axis_name="c", subcore_axis_name="s")
    cp = pltpu.CompilerParams()
    if "needs_layout_passes" in pltpu.CompilerParams.__dataclass_fields__:
        cp = dataclasses.replace(cp, needs_layout_passes=False)
    kern = functools.partial(
        pl.kernel,
        out_type=jax.ShapeDtypeStruct((rows, _K), jnp.int32),
        mesh=mesh,
        compiler_params=cp,
        scratch_types=[
            pltpu.VMEM((_WN,), jnp.float32),
            pltpu.VMEM((256,), jnp.float32),
            pltpu.VMEM((_K,), jnp.int32),
            pltpu.SemaphoreType.DMA,
        ],
    )(functools.partial(_sc_select, rows // _NW))
    return kern(dist)


def kernel(ref_bxyz, query_bxyz):
    m = query_bxyz.shape[0]
    n = ref_bxyz.shape[0]
    rt = jnp.transpose(ref_bxyz)
    rt = jnp.concatenate(
        [rt, jnp.full((4, _WPAD - n), 1e9, jnp.float32)], axis=1)
    rt = jnp.concatenate(
        [rt, jnp.zeros((4, _WPAD), jnp.float32)], axis=0)

    q_tc = query_bxyz[:_SPLIT]
    q_sc = query_bxyz[_SPLIT:]
    rows_sc = m - _SPLIT

    dist, s0 = pl.pallas_call(
        _tc_dist_block,
        grid=(rows_sc // _QB,),
        in_specs=[
            pl.BlockSpec((_QB, 4), lambda i: (i, 0)),
            pl.BlockSpec((8, _WPAD), lambda i: (0, 0)),
        ],
        out_specs=[
            pl.BlockSpec((_QB, _WN), lambda i: (i, 0)),
            pl.BlockSpec((_QB, 1), lambda i: (i, 0)),
        ],
        out_shape=[
            jax.ShapeDtypeStruct((rows_sc, _WN), jnp.float32),
            jax.ShapeDtypeStruct((rows_sc, 1), jnp.int32),
        ],
    )(q_sc, rt)
    loc = _sc_select_call(dist)
    e_sc = loc + s0

    e_tc = pl.pallas_call(
        _tc_extract_block,
        grid=(_SPLIT // _QB,),
        in_specs=[
            pl.BlockSpec((_QB, 4), lambda i: (i, 0)),
            pl.BlockSpec((8, _WPAD), lambda i: (0, 0)),
        ],
        out_specs=pl.BlockSpec((_QB, _K), lambda i: (i, 0)),
        out_shape=jax.ShapeDtypeStruct((_SPLIT, _K), jnp.int32),
        scratch_shapes=[pltpu.VMEM((_QB, _WS), jnp.float32)],
    )(q_tc, rt)

    e_ref = jnp.concatenate([e_tc, e_sc], axis=0).reshape(-1)
    e_query = jnp.broadcast_to(
        jnp.arange(m, dtype=jnp.int32)[:, None], (m, _K)).reshape(-1)
    return (e_ref, e_query)


# speedup vs baseline: 2.0016x; 1.0307x over previous
import dataclasses
import functools

import jax
import jax.numpy as jnp
from jax import lax
from jax.experimental import pallas as pl
from jax.experimental.pallas import tpu as pltpu
from jax.experimental.pallas import tpu_sc as plsc

_K = 32
_QB = 256
_WN = 2560
_WS = 4608
_WPAD = 12800
_M = 8192
_SPLIT = 4096
_NW = 32
_NCH = _WN // 16
_NSUP = _NCH // 16


def _window(width, s0, q_parts, r_ref):
    qb, qx, qy, qz = q_parts
    rb = r_ref[0:1, pl.ds(s0, width)]
    rx = r_ref[1:2, pl.ds(s0, width)]
    ry = r_ref[2:3, pl.ds(s0, width)]
    rz = r_ref[3:4, pl.ds(s0, width)]
    q2 = qx * qx + qy * qy + qz * qz
    r2 = rx * rx + ry * ry + rz * rz
    bf = jnp.bfloat16
    qmat = jnp.concatenate([qx, qy, qz], axis=1).astype(bf)
    rmat = jnp.concatenate([rx, ry, rz], axis=0).astype(bf)
    qr = jnp.dot(qmat, rmat, preferred_element_type=jnp.float32)
    dist = (q2 + r2) - 2.0 * qr
    return jnp.where(qb != rb, jnp.float32(1e30), dist)


def _bounds(qb, r_ref):
    b_lo = jnp.min(qb)
    b_hi = jnp.max(qb)
    rb_full = r_ref[0:1, :]
    r_lo = jnp.sum((rb_full < b_lo).astype(jnp.int32))
    r_hi = jnp.sum((rb_full <= b_hi).astype(jnp.int32))
    s0 = (r_lo // 128) * 128
    return b_hi, rb_full, r_hi, s0


def _extract(width, s0, q_parts, r_ref, o_ref, d_ref):
    dist = _window(width, s0, q_parts, r_ref)
    d_ref[:, 0:width] = dist

    iota = jax.lax.broadcasted_iota(jnp.int32, (1, width), 1)
    lane = jax.lax.broadcasted_iota(jnp.int32, (1, _K), 1)

    def body(k, best):
        dmat = d_ref[:, 0:width]
        m = jnp.min(dmat, axis=1, keepdims=True)
        isel = jnp.min(
            jnp.where(dmat == m, iota, jnp.int32(2**31 - 1)),
            axis=1, keepdims=True)
        d_ref[:, 0:width] = jnp.where(
            iota == isel, jnp.float32(jnp.inf), dmat)
        return jnp.where(lane == k, isel, best)

    best = jax.lax.fori_loop(
        0, _K, body, jnp.zeros((_QB, _K), jnp.int32))
    o_ref[...] = best + s0


def _tc_extract_block(q_ref, r_ref, o_ref, d_ref):
    q = q_ref[...]
    qb = q[:, 0:1]
    q_parts = (qb, q[:, 1:2], q[:, 2:3], q[:, 3:4])
    _, _, r_hi, s0 = _bounds(qb, r_ref)
    fits = (r_hi - s0) <= _WN

    @pl.when(fits)
    def _narrow():
        _extract(_WN, s0, q_parts, r_ref, o_ref, d_ref)

    @pl.when(jnp.logical_not(fits))
    def _wide():
        _extract(_WS, s0, q_parts, r_ref, o_ref, d_ref)


def _tc_dist_block(q_ref, r_ref, d_ref, s_ref):
    q = q_ref[...]
    qb = q[:, 0:1]
    q_parts = (qb, q[:, 1:2], q[:, 2:3], q[:, 3:4])
    b_hi, rb_full, r_hi, s0 = _bounds(qb, r_ref)
    fits = (r_hi - s0) <= _WN

    @pl.when(fits)
    def _narrow():
        d_ref[...] = _window(_WN, s0, q_parts, r_ref)
        s_ref[...] = jnp.full((_QB, 1), 0, jnp.int32) + s0

    @pl.when(jnp.logical_not(fits))
    def _wide():
        r_mid = jnp.sum((rb_full < b_hi).astype(jnp.int32))
        s1 = (r_mid // 128) * 128
        w_lo = _window(_WN, s0, q_parts, r_ref)
        w_hi = _window(_WN, s1, q_parts, r_ref)
        hi_row = qb == b_hi
        d_ref[...] = jnp.where(hi_row, w_hi, w_lo)
        s_ref[...] = jnp.where(hi_row, s1, s0) + jnp.full(
            (_QB, 1), 0, jnp.int32)


def _sc_select(rpw, d_hbm, o_hbm, rowbuf0, rowbuf1, gmin, outbuf, sem0, sem1):
    wid = lax.axis_index("s") * 2 + lax.axis_index("c")
    base = wid * rpw
    lanes = jax.lax.broadcasted_iota(jnp.int32, (16,), 0)
    inf16 = jnp.full((16,), jnp.inf, jnp.float32)

    for g in range(_NCH // 16, 16):
        gmin[pl.ds(16 * g, 16)] = inf16

    def _tree_min(rowbuf):
        for g in range(_NSUP):
            accs = [inf16, inf16, inf16, inf16]
            for j in range(16):
                v = plsc.load_gather(rowbuf, [256 * g + 16 * lanes + j])
                accs[j % 4] = jnp.minimum(accs[j % 4], v)
            gmin[pl.ds(16 * g, 16)] = jnp.minimum(
                jnp.minimum(accs[0], accs[1]), jnp.minimum(accs[2], accs[3]))

        accs = [inf16, inf16, inf16, inf16]
        for j in range(16):
            v = plsc.load_gather(gmin, [16 * lanes + j])
            accs[j % 4] = jnp.minimum(accs[j % 4], v)
        return jnp.minimum(
            jnp.minimum(accs[0], accs[1]), jnp.minimum(accs[2], accs[3]))

    def _process_row(rowbuf, row):
        smin = _tree_min(rowbuf)

        def round_(k, carry):
            smin, o0, o1 = carry
            m = jnp.min(smin)
            l_star = jnp.max(plsc.all_reduce_ffs(smin == m))
            gvec = gmin[pl.ds(16 * l_star, 16)]
            c_off = jnp.max(plsc.all_reduce_ffs(gvec == m))
            c_star = 16 * l_star + c_off
            cv = rowbuf[pl.ds(16 * c_star, 16)]
            j_star = jnp.max(plsc.all_reduce_ffs(cv == m))
            idx = jnp.full((16,), 16 * c_star + j_star, jnp.int32)
            first = k < 16
            hit = lanes == jnp.where(first, k, k - 16)
            o0 = jnp.where(hit & first, idx, o0)
            o1 = jnp.where(hit & jnp.logical_not(first), idx, o1)
            cv2 = jnp.where(lanes == j_star, jnp.float32(jnp.inf), cv)
            rowbuf[pl.ds(16 * c_star, 16)] = cv2
            ng = jnp.min(cv2)
            gv2 = jnp.where(lanes == c_off, ng, gvec)
            gmin[pl.ds(16 * l_star, 16)] = gv2
            ns = jnp.min(gv2)
            return (jnp.where(lanes == l_star, ns, smin), o0, o1)

        zero16 = jnp.zeros((16,), jnp.int32)
        _, o0, o1 = lax.fori_loop(0, _K, round_, (smin, zero16, zero16))
        outbuf[pl.ds(0, 16)] = o0
        outbuf[pl.ds(16, 16)] = o1
        pltpu.sync_copy(outbuf, o_hbm.at[row])

    pltpu.async_copy(d_hbm.at[base], rowbuf0, sem0)

    @pl.loop(0, rpw // 2)
    def _pair(i):
        r0 = base + 2 * i
        pltpu.make_async_copy(d_hbm.at[r0], rowbuf0, sem0).wait()
        pltpu.async_copy(d_hbm.at[r0 + 1], rowbuf1, sem1)
        _process_row(rowbuf0, r0)
        pltpu.make_async_copy(d_hbm.at[r0 + 1], rowbuf1, sem1).wait()

        @pl.when(2 * i + 2 < rpw)
        def _prefetch():
            pltpu.async_copy(d_hbm.at[r0 + 2], rowbuf0, sem0)

        _process_row(rowbuf1, r0 + 1)


def _sc_select_call(dist):
    rows = dist.shape[0]
    mesh = plsc.VectorSubcoreMesh(core_axis_name="c", subcore_axis_name="s")
    cp = pltpu.CompilerParams()
    if "needs_layout_passes" in pltpu.CompilerParams.__dataclass_fields__:
        cp = dataclasses.replace(cp, needs_layout_passes=False)
    kern = functools.partial(
        pl.kernel,
        out_type=jax.ShapeDtypeStruct((rows, _K), jnp.int32),
        mesh=mesh,
        compiler_params=cp,
        scratch_types=[
            pltpu.VMEM((_WN,), jnp.float32),
            pltpu.VMEM((_WN,), jnp.float32),
            pltpu.VMEM((256,), jnp.float32),
            pltpu.VMEM((_K,), jnp.int32),
            pltpu.SemaphoreType.DMA,
            pltpu.SemaphoreType.DMA,
        ],
    )(functools.partial(_sc_select, rows // _NW))
    return kern(dist)


def kernel(ref_bxyz, query_bxyz):
    m = query_bxyz.shape[0]
    n = ref_bxyz.shape[0]
    rt = jnp.transpose(ref_bxyz)
    rt = jnp.concatenate(
        [rt, jnp.full((4, _WPAD - n), 1e9, jnp.float32)], axis=1)
    rt = jnp.concatenate(
        [rt, jnp.zeros((4, _WPAD), jnp.float32)], axis=0)

    q_tc = query_bxyz[:_SPLIT]
    q_sc = query_bxyz[_SPLIT:]
    rows_sc = m - _SPLIT

    dist, s0 = pl.pallas_call(
        _tc_dist_block,
        grid=(rows_sc // _QB,),
        in_specs=[
            pl.BlockSpec((_QB, 4), lambda i: (i, 0)),
            pl.BlockSpec((8, _WPAD), lambda i: (0, 0)),
        ],
        out_specs=[
            pl.BlockSpec((_QB, _WN), lambda i: (i, 0)),
            pl.BlockSpec((_QB, 1), lambda i: (i, 0)),
        ],
        out_shape=[
            jax.ShapeDtypeStruct((rows_sc, _WN), jnp.float32),
            jax.ShapeDtypeStruct((rows_sc, 1), jnp.int32),
        ],
    )(q_sc, rt)
    loc = _sc_select_call(dist)
    e_sc = loc + s0

    e_tc = pl.pallas_call(
        _tc_extract_block,
        grid=(_SPLIT // _QB,),
        in_specs=[
            pl.BlockSpec((_QB, 4), lambda i: (i, 0)),
            pl.BlockSpec((8, _WPAD), lambda i: (0, 0)),
        ],
        out_specs=pl.BlockSpec((_QB, _K), lambda i: (i, 0)),
        out_shape=jax.ShapeDtypeStruct((_SPLIT, _K), jnp.int32),
        scratch_shapes=[pltpu.VMEM((_QB, _WS), jnp.float32)],
    )(q_tc, rt)

    e_ref = jnp.concatenate([e_tc, e_sc], axis=0).reshape(-1)
    e_query = jnp.broadcast_to(
        jnp.arange(m, dtype=jnp.int32)[:, None], (m, _K)).reshape(-1)
    return (e_ref, e_query)


# speedup vs baseline: 2.0988x; 1.0486x over previous
import dataclasses
import functools

import jax
import jax.numpy as jnp
from jax import lax
from jax.experimental import pallas as pl
from jax.experimental.pallas import tpu as pltpu
from jax.experimental.pallas import tpu_sc as plsc

_K = 32
_QB = 256
_WN = 2560
_WS = 4608
_WPAD = 12800
_M = 8192
_SPLIT = 3840
_NW = 32
_NCH = _WN // 16
_NSUP = _NCH // 16


def _window(width, s0, q_parts, r_ref):
    qb, qx, qy, qz = q_parts
    rb = r_ref[0:1, pl.ds(s0, width)]
    rx = r_ref[1:2, pl.ds(s0, width)]
    ry = r_ref[2:3, pl.ds(s0, width)]
    rz = r_ref[3:4, pl.ds(s0, width)]
    q2 = qx * qx + qy * qy + qz * qz
    r2 = rx * rx + ry * ry + rz * rz
    bf = jnp.bfloat16
    qmat = jnp.concatenate([qx, qy, qz], axis=1).astype(bf)
    rmat = jnp.concatenate([rx, ry, rz], axis=0).astype(bf)
    qr = jnp.dot(qmat, rmat, preferred_element_type=jnp.float32)
    dist = (q2 + r2) - 2.0 * qr
    return jnp.where(qb != rb, jnp.float32(1e30), dist)


def _bounds(qb, r_ref):
    b_lo = jnp.min(qb)
    b_hi = jnp.max(qb)
    rb_full = r_ref[0:1, :]
    r_lo = jnp.sum((rb_full < b_lo).astype(jnp.int32))
    r_hi = jnp.sum((rb_full <= b_hi).astype(jnp.int32))
    s0 = (r_lo // 128) * 128
    return b_hi, rb_full, r_hi, s0


def _extract(width, s0, q_parts, r_ref, o_ref, d_ref):
    dist = _window(width, s0, q_parts, r_ref)
    d_ref[:, 0:width] = dist

    iota = jax.lax.broadcasted_iota(jnp.int32, (1, width), 1)
    lane = jax.lax.broadcasted_iota(jnp.int32, (1, _K), 1)

    def body(k, best):
        dmat = d_ref[:, 0:width]
        m = jnp.min(dmat, axis=1, keepdims=True)
        isel = jnp.min(
            jnp.where(dmat == m, iota, jnp.int32(2**31 - 1)),
            axis=1, keepdims=True)
        d_ref[:, 0:width] = jnp.where(
            iota == isel, jnp.float32(jnp.inf), dmat)
        return jnp.where(lane == k, isel, best)

    best = jax.lax.fori_loop(
        0, _K, body, jnp.zeros((_QB, _K), jnp.int32))
    o_ref[...] = best + s0


def _tc_extract_block(q_ref, r_ref, o_ref, d_ref):
    q = q_ref[...]
    qb = q[:, 0:1]
    q_parts = (qb, q[:, 1:2], q[:, 2:3], q[:, 3:4])
    _, _, r_hi, s0 = _bounds(qb, r_ref)
    fits = (r_hi - s0) <= _WN

    @pl.when(fits)
    def _narrow():
        _extract(_WN, s0, q_parts, r_ref, o_ref, d_ref)

    @pl.when(jnp.logical_not(fits))
    def _wide():
        _extract(_WS, s0, q_parts, r_ref, o_ref, d_ref)


def _tc_dist_block(q_ref, r_ref, d_ref, s_ref):
    q = q_ref[...]
    qb = q[:, 0:1]
    q_parts = (qb, q[:, 1:2], q[:, 2:3], q[:, 3:4])
    b_hi, rb_full, r_hi, s0 = _bounds(qb, r_ref)
    fits = (r_hi - s0) <= _WN

    @pl.when(fits)
    def _narrow():
        d_ref[...] = _window(_WN, s0, q_parts, r_ref)
        s_ref[...] = jnp.full((_QB, 1), 0, jnp.int32) + s0

    @pl.when(jnp.logical_not(fits))
    def _wide():
        r_mid = jnp.sum((rb_full < b_hi).astype(jnp.int32))
        s1 = (r_mid // 128) * 128
        w_lo = _window(_WN, s0, q_parts, r_ref)
        w_hi = _window(_WN, s1, q_parts, r_ref)
        hi_row = qb == b_hi
        d_ref[...] = jnp.where(hi_row, w_hi, w_lo)
        s_ref[...] = jnp.where(hi_row, s1, s0) + jnp.full(
            (_QB, 1), 0, jnp.int32)


def _sc_select(rpw, d_hbm, o_hbm, rowbuf0, rowbuf1, gmin, outbuf, sem0, sem1):
    wid = lax.axis_index("s") * 2 + lax.axis_index("c")
    base = wid * rpw
    lanes = jax.lax.broadcasted_iota(jnp.int32, (16,), 0)
    inf16 = jnp.full((16,), jnp.inf, jnp.float32)

    for g in range(_NCH // 16, 16):
        gmin[pl.ds(16 * g, 16)] = inf16

    def _tree_min(rowbuf):
        for g in range(_NSUP):
            accs = [inf16, inf16, inf16, inf16]
            for j in range(16):
                v = plsc.load_gather(rowbuf, [256 * g + 16 * lanes + j])
                accs[j % 4] = jnp.minimum(accs[j % 4], v)
            gmin[pl.ds(16 * g, 16)] = jnp.minimum(
                jnp.minimum(accs[0], accs[1]), jnp.minimum(accs[2], accs[3]))

        accs = [inf16, inf16, inf16, inf16]
        for j in range(16):
            v = plsc.load_gather(gmin, [16 * lanes + j])
            accs[j % 4] = jnp.minimum(accs[j % 4], v)
        return jnp.minimum(
            jnp.minimum(accs[0], accs[1]), jnp.minimum(accs[2], accs[3]))

    def _process_row(rowbuf, row):
        smin = _tree_min(rowbuf)

        def round_(k, carry):
            smin, o0, o1 = carry
            m = jnp.min(smin)
            l_star = jnp.max(plsc.all_reduce_ffs(smin == m))
            gvec = gmin[pl.ds(16 * l_star, 16)]
            c_off = jnp.max(plsc.all_reduce_ffs(gvec == m))
            c_star = 16 * l_star + c_off
            cv = rowbuf[pl.ds(16 * c_star, 16)]
            j_star = jnp.max(plsc.all_reduce_ffs(cv == m))
            idx = jnp.full((16,), 16 * c_star + j_star, jnp.int32)
            first = k < 16
            hit = lanes == jnp.where(first, k, k - 16)
            o0 = jnp.where(hit & first, idx, o0)
            o1 = jnp.where(hit & jnp.logical_not(first), idx, o1)
            cv2 = jnp.where(lanes == j_star, jnp.float32(jnp.inf), cv)
            rowbuf[pl.ds(16 * c_star, 16)] = cv2
            ng = jnp.min(cv2)
            gv2 = jnp.where(lanes == c_off, ng, gvec)
            gmin[pl.ds(16 * l_star, 16)] = gv2
            ns = jnp.min(gv2)
            return (jnp.where(lanes == l_star, ns, smin), o0, o1)

        zero16 = jnp.zeros((16,), jnp.int32)
        _, o0, o1 = lax.fori_loop(0, _K, round_, (smin, zero16, zero16))
        outbuf[pl.ds(0, 16)] = o0
        outbuf[pl.ds(16, 16)] = o1
        pltpu.sync_copy(outbuf, o_hbm.at[row])

    pltpu.async_copy(d_hbm.at[base], rowbuf0, sem0)

    @pl.loop(0, rpw // 2)
    def _pair(i):
        r0 = base + 2 * i
        pltpu.make_async_copy(d_hbm.at[r0], rowbuf0, sem0).wait()
        pltpu.async_copy(d_hbm.at[r0 + 1], rowbuf1, sem1)
        _process_row(rowbuf0, r0)
        pltpu.make_async_copy(d_hbm.at[r0 + 1], rowbuf1, sem1).wait()

        @pl.when(2 * i + 2 < rpw)
        def _prefetch():
            pltpu.async_copy(d_hbm.at[r0 + 2], rowbuf0, sem0)

        _process_row(rowbuf1, r0 + 1)


def _sc_select_call(dist):
    rows = dist.shape[0]
    mesh = plsc.VectorSubcoreMesh(core_axis_name="c", subcore_axis_name="s")
    cp = pltpu.CompilerParams()
    if "needs_layout_passes" in pltpu.CompilerParams.__dataclass_fields__:
        cp = dataclasses.replace(cp, needs_layout_passes=False)
    kern = functools.partial(
        pl.kernel,
        out_type=jax.ShapeDtypeStruct((rows, _K), jnp.int32),
        mesh=mesh,
        compiler_params=cp,
        scratch_types=[
            pltpu.VMEM((_WN,), jnp.float32),
            pltpu.VMEM((_WN,), jnp.float32),
            pltpu.VMEM((256,), jnp.float32),
            pltpu.VMEM((_K,), jnp.int32),
            pltpu.SemaphoreType.DMA,
            pltpu.SemaphoreType.DMA,
        ],
    )(functools.partial(_sc_select, rows // _NW))
    return kern(dist)


def kernel(ref_bxyz, query_bxyz):
    m = query_bxyz.shape[0]
    n = ref_bxyz.shape[0]
    rt = jnp.transpose(ref_bxyz)
    rt = jnp.concatenate(
        [rt, jnp.full((4, _WPAD - n), 1e9, jnp.float32)], axis=1)
    rt = jnp.concatenate(
        [rt, jnp.zeros((4, _WPAD), jnp.float32)], axis=0)

    q_tc = query_bxyz[:_SPLIT]
    q_sc = query_bxyz[_SPLIT:]
    rows_sc = m - _SPLIT

    dist, s0 = pl.pallas_call(
        _tc_dist_block,
        grid=(rows_sc // _QB,),
        in_specs=[
            pl.BlockSpec((_QB, 4), lambda i: (i, 0)),
            pl.BlockSpec((8, _WPAD), lambda i: (0, 0)),
        ],
        out_specs=[
            pl.BlockSpec((_QB, _WN), lambda i: (i, 0)),
            pl.BlockSpec((_QB, 1), lambda i: (i, 0)),
        ],
        out_shape=[
            jax.ShapeDtypeStruct((rows_sc, _WN), jnp.float32),
            jax.ShapeDtypeStruct((rows_sc, 1), jnp.int32),
        ],
    )(q_sc, rt)
    loc = _sc_select_call(dist)
    e_sc = loc + s0

    e_tc = pl.pallas_call(
        _tc_extract_block,
        grid=(_SPLIT // _QB,),
        in_specs=[
            pl.BlockSpec((_QB, 4), lambda i: (i, 0)),
            pl.BlockSpec((8, _WPAD), lambda i: (0, 0)),
        ],
        out_specs=pl.BlockSpec((_QB, _K), lambda i: (i, 0)),
        out_shape=jax.ShapeDtypeStruct((_SPLIT, _K), jnp.int32),
        scratch_shapes=[pltpu.VMEM((_QB, _WS), jnp.float32)],
    )(q_tc, rt)

    e_ref = jnp.concatenate([e_tc, e_sc], axis=0).reshape(-1)
    e_query = jnp.broadcast_to(
        jnp.arange(m, dtype=jnp.int32)[:, None], (m, _K)).reshape(-1)
    return (e_ref, e_query)
